# Initial kernel scaffold; baseline (speedup 1.0000x reference)
import jax
import jax.numpy as jnp
from jax.experimental import pallas as pl


def kernel(locs, raster):
    raise NotImplementedError("write your pallas kernel here")


# speedup vs baseline: 1.3574x; 1.3574x over previous
import functools

import jax
import jax.numpy as jnp
from jax import lax
from jax.experimental import pallas as pl
from jax.experimental.pallas import tpu as pltpu
from jax.experimental.pallas import tpu_sc as plsc

_H, _W, _C = 1080, 2160, 20
_B = 16384
_NC, _NS, _L = 2, 16, 16
_NW = _NC * _NS
_BPW = _B // _NW
_NG = _BPW // _L
_NB = 32
_NBATCH = _BPW // _NB
_RPB = 4 * _NB


def _bcast_lane(vec, lane_idx):
    dn = lax.GatherDimensionNumbers(offset_dims=(), collapsed_slice_dims=(0,),
                                    start_index_map=(0,))
    return lax.gather(vec, lane_idx[:, None], dn, (1,),
                      mode=lax.GatherScatterMode.PROMISE_IN_BOUNDS)


def _floor_parts(v):
    i = v.astype(jnp.int32)
    f = i.astype(jnp.float32)
    adj = f > v
    return jnp.where(adj, i - 1, i), jnp.where(adj, f - 1.0, f)


def _sc_body(lon_hbm, lat_hbm, tab_hbm, out_hbm,
             lon_v, lat_v, idx_v, w_v, rows0, rows1, out_v, sem0, sem1):
    wid = lax.axis_index("s") * _NC + lax.axis_index("c")
    base = wid * _BPW

    pltpu.sync_copy(lon_hbm.at[pl.ds(base, _BPW)], lon_v)
    pltpu.sync_copy(lat_hbm.at[pl.ds(base, _BPW)], lat_v)

    def phase1(i, carry):
        lonv = lon_v[pl.ds(i * _L, _L)]
        latv = lat_v[pl.ds(i * _L, _L)]
        lonn = ((lonv + 180.0) / 360.0) * 2.0 - 1.0
        latn = ((latv + 90.0) / 180.0) * 2.0 - 1.0
        xh = (lonn + 1.0) / 2.0
        yh = 1.0 - (latn + 1.0) / 2.0
        xs = xh * float(_W) - 1.0
        ys = yh * float(_H) - 1.0
        xi, xf = _floor_parts(xs)
        yi, yf = _floor_parts(ys)
        dx = xs - xf
        dy = ys - yf
        xxp = jnp.minimum(xi + 1, _W - 1)
        yyp = jnp.minimum(yi + 1, _H - 1)
        xx = jnp.where(xi < 0, xi + _W, xi)
        yy = jnp.where(yi < 0, yi + _H, yi)
        r0 = yy * _W
        r1 = yyp * _W
        flats = (r0 + xx, r0 + xxp, r1 + xx, r1 + xxp)
        omdx = 1.0 - dx
        omdy = 1.0 - dy
        ws = (omdx * omdy, dx * omdy, omdx * dy, dx * dy)
        slot0 = (i // 2) * (4 * _NB) + (i % 2) * _L
        for k in range(4):
            idx_v[pl.ds(slot0 + k * _NB, _L)] = flats[k]
            w_v[pl.ds(slot0 + k * _NB, _L)] = ws[k]
        return carry

    lax.fori_loop(0, _NG, phase1, 0)

    def issue(n, buf, sem):
        def issue_group(g, carry):
            idxvec = idx_v[pl.ds(n * _RPB + g * _L, _L)]
            for lane in range(_L):
                r = idxvec[lane]
                pltpu.make_async_copy(tab_hbm.at[pl.ds(r, 1)],
                                      buf.at[pl.ds(g * _L + lane, 1)],
                                      sem).start()
            return carry
        lax.fori_loop(0, _RPB // _L, issue_group, 0)

    def drain(buf, sem):
        pltpu.make_async_copy(tab_hbm.at[pl.ds(0, _RPB)], buf, sem).wait()

    def compute(n, buf):
        for h in range(2):
            wvecs = [w_v[pl.ds(n * _RPB + k * _NB + h * _L, _L)]
                     for k in range(4)]
            for lane in range(_L):
                lb = h * _L + lane
                lane_idx = jnp.full((_L,), lane, jnp.int32)
                wb = [_bcast_lane(wvecs[k], lane_idx) for k in range(4)]
                for off in (0, _C - _L):
                    acc = wb[0] * buf[lb, pl.ds(off, _L)]
                    for k in range(1, 4):
                        acc = acc + wb[k] * buf[k * _NB + lb, pl.ds(off, _L)]
                    out_v[n * _NB + lb, pl.ds(off, _L)] = acc

    issue(0, rows0, sem0)
    issue(1, rows1, sem1)

    def pair(m, carry):
        for p in range(2):
            n = 2 * m + p
            buf = rows0 if p == 0 else rows1
            sem = sem0 if p == 0 else sem1
            drain(buf, sem)
            compute(n, buf)

            @pl.when(n + 2 < _NBATCH)
            def _():
                issue(n + 2, buf, sem)
        return carry

    lax.fori_loop(0, _NBATCH // 2, pair, 0)

    pltpu.sync_copy(out_v, out_hbm.at[pl.ds(base, _BPW)])


@jax.jit
def _sc_bilinear(lon, lat, table):
    mesh = plsc.VectorSubcoreMesh(core_axis_name="c", subcore_axis_name="s")
    fn = functools.partial(
        pl.kernel,
        mesh=mesh,
        out_type=jax.ShapeDtypeStruct((_B, _C), jnp.float32),
        scratch_types=[
            pltpu.VMEM((_BPW,), jnp.float32),
            pltpu.VMEM((_BPW,), jnp.float32),
            pltpu.VMEM((4 * _BPW,), jnp.int32),
            pltpu.VMEM((4 * _BPW,), jnp.float32),
            pltpu.VMEM((_RPB, _C), jnp.float32),
            pltpu.VMEM((_RPB, _C), jnp.float32),
            pltpu.VMEM((_BPW, _C), jnp.float32),
            pltpu.SemaphoreType.DMA,
            pltpu.SemaphoreType.DMA,
        ],
    )(_sc_body)
    return fn(lon, lat, table)


def kernel(locs, raster):
    lon = locs[:, 0]
    lat = locs[:, 1]
    table = raster.reshape(_H * _W, _C)
    return _sc_bilinear(lon, lat, table)


# speedup vs baseline: 4.6664x; 3.4378x over previous
import functools

import jax
import jax.numpy as jnp
from jax import lax
from jax.experimental import pallas as pl
from jax.experimental.pallas import tpu as pltpu
from jax.experimental.pallas import tpu_sc as plsc

_H, _W, _C = 1080, 2160, 20
_PLANE = _H * _W
_B = 16384
_NC, _NS, _L = 2, 16, 16
_NW = _NC * _NS
_BPW = _B // _NW
_NG = _BPW // _L
_NB = 32
_NBATCH = _BPW // _NB
_CPB = 4 * _NB


def _floor_parts(v):
    i = v.astype(jnp.int32)
    f = i.astype(jnp.float32)
    adj = f > v
    return jnp.where(adj, i - 1, i), jnp.where(adj, f - 1.0, f)


def _sc_body(lon_hbm, lat_hbm, tab_hbm, out_hbm,
             lon_v, lat_v, idx_v, w_v, cols0, cols1, out_v, sem0, sem1):
    wid = lax.axis_index("s") * _NC + lax.axis_index("c")
    base = wid * _BPW

    pltpu.sync_copy(lon_hbm.at[pl.ds(base, _BPW)], lon_v)
    pltpu.sync_copy(lat_hbm.at[pl.ds(base, _BPW)], lat_v)

    def phase1(i, carry):
        lonv = lon_v[pl.ds(i * _L, _L)]
        latv = lat_v[pl.ds(i * _L, _L)]
        lonn = ((lonv + 180.0) / 360.0) * 2.0 - 1.0
        latn = ((latv + 90.0) / 180.0) * 2.0 - 1.0
        xh = (lonn + 1.0) / 2.0
        yh = 1.0 - (latn + 1.0) / 2.0
        xs = xh * float(_W) - 1.0
        ys = yh * float(_H) - 1.0
        xi, xf = _floor_parts(xs)
        yi, yf = _floor_parts(ys)
        dx = xs - xf
        dy = ys - yf
        xxp = jnp.minimum(xi + 1, _W - 1)
        yyp = jnp.minimum(yi + 1, _H - 1)
        xx = jnp.where(xi < 0, xi + _W, xi)
        yy = jnp.where(yi < 0, yi + _H, yi)
        r0 = yy * _W
        r1 = yyp * _W
        flats = (r0 + xx, r0 + xxp, r1 + xx, r1 + xxp)
        omdx = 1.0 - dx
        omdy = 1.0 - dy
        ws = (omdx * omdy, dx * omdy, omdx * dy, dx * dy)
        n = i // 2
        half = (i % 2) * _L
        for k in range(4):
            idx_v[n, pl.ds(k * _NB + half, _L)] = flats[k]
            w_v[pl.ds(n * _CPB + k * _NB + half, _L)] = ws[k]
        return carry

    lax.fori_loop(0, _NG, phase1, 0)

    def issue(n, buf, sem):
        idx = idx_v.at[n]
        for c in range(_C):
            pltpu.async_copy(tab_hbm.at[pl.ds(c * _PLANE, _PLANE)].at[idx],
                             buf.at[c], sem)

    def drain(buf, sem):
        pltpu.make_async_copy(out_hbm.at[:, pl.ds(0, _CPB)], buf, sem).wait()

    def compute(n, buf):
        wvecs = [[w_v[pl.ds(n * _CPB + k * _NB + h * _L, _L)]
                  for h in range(2)] for k in range(4)]
        for c in range(_C):
            for h in range(2):
                acc = wvecs[0][h] * buf[c, pl.ds(h * _L, _L)]
                for k in range(1, 4):
                    acc = acc + wvecs[k][h] * buf[c, pl.ds(k * _NB + h * _L, _L)]
                out_v[c, pl.ds(n * _NB + h * _L, _L)] = acc

    issue(0, cols0, sem0)
    issue(1, cols1, sem1)

    def pair(m, carry):
        for p in range(2):
            n = 2 * m + p
            buf = cols0 if p == 0 else cols1
            sem = sem0 if p == 0 else sem1
            drain(buf, sem)
            compute(n, buf)

            @pl.when(n + 2 < _NBATCH)
            def _():
                issue(n + 2, buf, sem)
        return carry

    lax.fori_loop(0, _NBATCH // 2, pair, 0)

    pltpu.sync_copy(out_v, out_hbm.at[:, pl.ds(base, _BPW)])


@jax.jit
def _sc_bilinear(lon, lat, table):
    mesh = plsc.VectorSubcoreMesh(core_axis_name="c", subcore_axis_name="s")
    fn = functools.partial(
        pl.kernel,
        mesh=mesh,
        out_type=jax.ShapeDtypeStruct((_C, _B), jnp.float32),
        scratch_types=[
            pltpu.VMEM((_BPW,), jnp.float32),
            pltpu.VMEM((_BPW,), jnp.float32),
            pltpu.VMEM((_NBATCH, _CPB), jnp.int32),
            pltpu.VMEM((4 * _BPW,), jnp.float32),
            pltpu.VMEM((_C, _CPB), jnp.float32),
            pltpu.VMEM((_C, _CPB), jnp.float32),
            pltpu.VMEM((_C, _BPW), jnp.float32),
            pltpu.SemaphoreType.DMA,
            pltpu.SemaphoreType.DMA,
        ],
    )(_sc_body)
    return fn(lon, lat, table)


def kernel(locs, raster):
    lon = locs[:, 0]
    lat = locs[:, 1]
    table = jnp.transpose(raster, (2, 0, 1)).reshape(-1)
    out_t = _sc_bilinear(lon, lat, table)
    return out_t.T
